# baseline (speedup 1.0000x reference)
import jax
import jax.numpy as jnp
from jax import lax
from jax.experimental import pallas as pl
from jax.experimental.pallas import tpu as pltpu

TOP_K = 2
AUX_COEF = 0.01
TM = 1024
PK = 20


def _router_body(x_ref, wt_ref, out_ref, aux_ref, acc_ref):
    step = pl.program_id(0)
    nsteps = pl.num_programs(0)
    E = wt_ref.shape[1]
    T_total = TM * nsteps

    logits = jnp.dot(x_ref[...], wt_ref[...], preferred_element_type=jnp.float32)

    m = jnp.max(logits, axis=-1, keepdims=True)
    e = jnp.exp(logits - m)
    s = jnp.sum(e, axis=-1, keepdims=True)
    p = e / s

    iota = lax.broadcasted_iota(jnp.int32, (TM, E), 1)
    idx1 = jnp.min(jnp.where(logits == m, iota, E), axis=-1, keepdims=True)
    mask1 = iota == idx1
    l2 = jnp.where(mask1, -jnp.inf, logits)
    m2 = jnp.max(l2, axis=-1, keepdims=True)
    idx2 = jnp.min(jnp.where(l2 == m2, iota, E), axis=-1, keepdims=True)

    p1 = jnp.sum(jnp.where(mask1, p, 0.0), axis=-1, keepdims=True)
    p2 = jnp.sum(jnp.where(iota == idx2, p, 0.0), axis=-1, keepdims=True)
    denom = p1 + p2

    out_ref[:, 0:E] = logits
    out_ref[:, E:E + 4] = jnp.concatenate(
        [p1 / denom, p2 / denom,
         idx1.astype(jnp.float32), idx2.astype(jnp.float32)], axis=1
    )

    f_part = jnp.sum(jnp.where(mask1, 1.0, 0.0), axis=0, keepdims=True)
    p_part = jnp.sum(p, axis=0, keepdims=True)

    @pl.when(step == 0)
    def _init():
        acc_ref[...] = jnp.zeros_like(acc_ref)

    acc_ref[0:1, :] += f_part
    acc_ref[1:2, :] += p_part

    @pl.when(step == nsteps - 1)
    def _finish():
        aux = (AUX_COEF * E / (float(T_total) * float(T_total))) * jnp.sum(
            acc_ref[0:1, :] * acc_ref[1:2, :]
        )
        aux_ref[...] = jnp.reshape(aux, (1, 1))


def kernel(hidden_states, W):
    T, H = hidden_states.shape
    E = W.shape[0]
    wt = W.T
    grid = (T // TM,)
    packed, aux = pl.pallas_call(
        _router_body,
        grid=grid,
        in_specs=[
            pl.BlockSpec((TM, H), lambda i: (i, 0)),
            pl.BlockSpec((H, E), lambda i: (0, 0)),
        ],
        out_specs=[
            pl.BlockSpec((TM, PK), lambda i: (i, 0)),
            pl.BlockSpec((1, 1), lambda i: (0, 0)),
        ],
        out_shape=[
            jax.ShapeDtypeStruct((T, PK), jnp.float32),
            jax.ShapeDtypeStruct((1, 1), jnp.float32),
        ],
        scratch_shapes=[pltpu.VMEM((2, E), jnp.float32)],
    )(hidden_states, wt)
    logits = packed[:, :E]
    rw = packed[:, E:E + TOP_K]
    sel = packed[:, E + TOP_K:E + 2 * TOP_K].astype(jnp.int32)
    return rw, sel, logits, aux[0, 0]


# speedup vs baseline: 1.1746x; 1.1746x over previous
import jax
import jax.numpy as jnp
from jax.experimental import pallas as pl
from jax.experimental.pallas import tpu as pltpu

TOP_K = 2
AUX_COEF = 0.01
TM = 1024


def _router_body(x_ref, wt_ref, rw_ref, sel_ref, logits_ref, aux_ref, acc_ref):
    i = pl.program_id(0)
    nsteps = pl.num_programs(0)
    E = wt_ref.shape[1]
    tm = x_ref.shape[0]
    T_total = tm * nsteps

    logits_ref[...] = jnp.dot(
        x_ref[...], wt_ref[...], preferred_element_type=jnp.float32
    )

    SUB = 128
    f_part = jnp.zeros((1, E), jnp.float32)
    p_part = jnp.zeros((1, E), jnp.float32)
    iota = jax.lax.broadcasted_iota(jnp.int32, (SUB, E), 1)
    for j in range(tm // SUB):
        rows = pl.ds(j * SUB, SUB)
        logits = logits_ref[rows, :]
        m = jnp.max(logits, axis=-1, keepdims=True)
        e = jnp.exp(logits - m)
        s = jnp.sum(e, axis=-1, keepdims=True)
        p = e / s

        idx1 = jnp.min(jnp.where(logits == m, iota, E), axis=-1, keepdims=True)
        l2 = jnp.where(iota == idx1, -jnp.inf, logits)
        m2 = jnp.max(l2, axis=-1, keepdims=True)
        idx2 = jnp.min(jnp.where(l2 == m2, iota, E), axis=-1, keepdims=True)

        p1 = jnp.sum(jnp.where(iota == idx1, p, 0.0), axis=-1, keepdims=True)
        p2 = jnp.sum(jnp.where(iota == idx2, p, 0.0), axis=-1, keepdims=True)
        denom = p1 + p2
        rw_ref[rows, :] = jnp.concatenate([p1 / denom, p2 / denom], axis=1)
        sel_ref[rows, :] = jnp.concatenate([idx1, idx2], axis=1)

        f_part = f_part + jnp.sum(
            jnp.where(iota == idx1, 1.0, 0.0), axis=0, keepdims=True
        )
        p_part = p_part + jnp.sum(p, axis=0, keepdims=True)

    @pl.when(i == 0)
    def _init():
        acc_ref[...] = jnp.zeros_like(acc_ref)

    acc_ref[0:1, :] += f_part
    acc_ref[1:2, :] += p_part

    @pl.when(i == nsteps - 1)
    def _finish():
        aux = (AUX_COEF * E / (T_total * T_total)) * jnp.sum(
            acc_ref[0:1, :] * acc_ref[1:2, :]
        )
        aux_ref[...] = jnp.reshape(aux, (1, 1))


def kernel(hidden_states, W):
    T, H = hidden_states.shape
    E = W.shape[0]
    wt = W.T
    grid = (T // TM,)
    rw, sel, logits, aux = pl.pallas_call(
        _router_body,
        grid=grid,
        in_specs=[
            pl.BlockSpec((TM, H), lambda i: (i, 0)),
            pl.BlockSpec((H, E), lambda i: (0, 0)),
        ],
        out_specs=[
            pl.BlockSpec((TM, TOP_K), lambda i: (i, 0)),
            pl.BlockSpec((TM, TOP_K), lambda i: (i, 0)),
            pl.BlockSpec((TM, E), lambda i: (i, 0)),
            pl.BlockSpec((1, 1), lambda i: (0, 0)),
        ],
        out_shape=[
            jax.ShapeDtypeStruct((T, TOP_K), jnp.float32),
            jax.ShapeDtypeStruct((T, TOP_K), jnp.int32),
            jax.ShapeDtypeStruct((T, E), jnp.float32),
            jax.ShapeDtypeStruct((1, 1), jnp.float32),
        ],
        scratch_shapes=[pltpu.VMEM((2, E), jnp.float32)],
    )(hidden_states, wt)
    return rw, sel, logits, aux[0, 0]
